# baseline (speedup 1.0000x reference)
import functools

import jax
import jax.numpy as jnp
from jax import lax
from jax.experimental import pallas as pl
from jax.experimental.pallas import tpu as pltpu
from jax.experimental.pallas import tpu_sc as plsc

B, N, S, NS, D = 2, 8192, 2048, 32, 16
C_IN = 3 + D
C1, C2 = 32, 64
EPS = 1e-5
BLK = 128
RBLK = 4096
QBLK = RBLK // NS
NBLK = (B * S * NS) // RBLK
BIG = 3.0e38
HIGH = lax.Precision.HIGHEST


def _k1_body(xyz_ref, pts_ref, w0t_ref, b0_ref, idx_ref, f1_ref, gidx_ref):
    b = pl.program_id(0)
    x = xyz_ref[0]
    p = pts_ref[0]
    w = w0t_ref[...]
    f = (jnp.dot(x, w[0:3, :], precision=HIGH, preferred_element_type=jnp.float32)
         + jnp.dot(p, w[3:, :], precision=HIGH, preferred_element_type=jnp.float32)
         + b0_ref[...])
    f1_ref[0] = f
    gidx_ref[0] = idx_ref[0] + b * N


def _run_k1(xyz, points, w0t, b0, idxr):
    return pl.pallas_call(
        _k1_body,
        grid=(B,),
        in_specs=[
            pl.BlockSpec((1, N, 3), lambda b: (b, 0, 0)),
            pl.BlockSpec((1, N, D), lambda b: (b, 0, 0)),
            pl.BlockSpec((C_IN, C1), lambda b: (0, 0)),
            pl.BlockSpec((1, C1), lambda b: (0, 0)),
            pl.BlockSpec((1, 1, S), lambda b: (b, 0, 0)),
        ],
        out_specs=[
            pl.BlockSpec((1, N, C1), lambda b: (b, 0, 0)),
            pl.BlockSpec((1, 1, S), lambda b: (b, 0, 0)),
        ],
        out_shape=[
            jax.ShapeDtypeStruct((B, N, C1), jnp.float32),
            jax.ShapeDtypeStruct((B, 1, S), jnp.int32),
        ],
    )(xyz, points, w0t, b0, idxr)


NSUB = 8
NLANE = N // NSUB


def _k2_body(xyzs_ref, q_ref, knn_ref):
    b = pl.program_id(0)
    q = q_ref[0]

    d3 = None
    for k in range(3):
        p = xyzs_ref[0, k]
        diff = q[:, k][:, None, None] - p[None]
        d3 = diff * diff if d3 is None else d3 + diff * diff

    n_iota = (lax.broadcasted_iota(jnp.int32, (BLK, NSUB, NLANE), 1) * NLANE
              + lax.broadcasted_iota(jnp.int32, (BLK, NSUB, NLANE), 2))

    cols = []
    for _ in range(NS):
        gmin = jnp.min(jnp.min(d3, axis=1, keepdims=True), axis=2,
                       keepdims=True)
        cand = jnp.where(d3 == gmin, n_iota, N)
        nstar = jnp.min(jnp.min(cand, axis=1, keepdims=True), axis=2,
                        keepdims=True)
        cols.append(nstar[:, 0, :])
        d3 = jnp.where(n_iota == nstar, BIG, d3)
    knn_ref[0] = jnp.concatenate(cols, axis=1) + b * N


def _run_k2(xyzs, nxyz):
    return pl.pallas_call(
        _k2_body,
        grid=(B, S // BLK),
        in_specs=[
            pl.BlockSpec((1, 3, NSUB, NLANE), lambda b, j: (b, 0, 0, 0)),
            pl.BlockSpec((1, BLK, 3), lambda b, j: (b, j, 0)),
        ],
        out_specs=pl.BlockSpec((1, BLK, NS), lambda b, j: (b, j, 0)),
        out_shape=jax.ShapeDtypeStruct((B, S, NS), jnp.int32),
    )(xyzs, nxyz)


_IDX_TOTAL = B * S * NS
_GCH = 128


def _make_sc_gather(n_idx, width):
    info = plsc.get_sparse_core_info()
    nw = info.num_cores * info.num_subcores
    per_w = n_idx // nw
    nch = per_w // _GCH
    mesh = plsc.VectorSubcoreMesh(core_axis_name="c", subcore_axis_name="s")

    @functools.partial(
        pl.kernel,
        mesh=mesh,
        compiler_params=pltpu.CompilerParams(use_tc_tiling_on_sc=False),
        out_type=jax.ShapeDtypeStruct((n_idx, width), jnp.float32),
        scratch_types=[
            pltpu.VMEM((per_w,), jnp.int32),
            pltpu.VMEM((_GCH, width), jnp.float32),
            pltpu.SemaphoreType.DMA,
        ],
    )
    def gather_k(table_hbm, idx_hbm, out_hbm, idx_v, buf0, sem0):
        wid = lax.axis_index("s") * info.num_cores + lax.axis_index("c")
        base = pl.multiple_of(wid * per_w, _GCH)
        pltpu.sync_copy(idx_hbm.at[pl.ds(base, per_w)], idx_v)

        def body(j, _):
            off = pl.multiple_of(j * _GCH, _GCH)
            pltpu.async_copy(
                table_hbm.at[idx_v.at[pl.ds(off, _GCH)]], buf0, sem0).wait()
            dst = pl.multiple_of(base + j * _GCH, _GCH)
            pltpu.sync_copy(buf0, out_hbm.at[pl.ds(dst, _GCH)])
            return 0

        lax.fori_loop(0, nch, body, 0)

    return gather_k


def _k4a_body(g_ref, nx_ref, w0t_ref, s1_ref, q1_ref):
    q = jnp.dot(nx_ref[...], w0t_ref[0:3, :], precision=HIGH,
                preferred_element_type=jnp.float32)
    z1 = g_ref[...].reshape(QBLK, NS, C1) - q[:, None, :]
    s = jnp.sum(z1, axis=(0, 1)).reshape(1, C1)
    sq = jnp.sum(z1 * z1, axis=(0, 1)).reshape(1, C1)

    @pl.when(pl.program_id(0) == 0)
    def _():
        s1_ref[...] = jnp.zeros_like(s1_ref)
        q1_ref[...] = jnp.zeros_like(q1_ref)

    s1_ref[...] += s
    q1_ref[...] += sq


def _run_k4a(gflat, nxflat, w0t):
    return pl.pallas_call(
        _k4a_body,
        grid=(NBLK,),
        in_specs=[
            pl.BlockSpec((RBLK, C1), lambda i: (i, 0)),
            pl.BlockSpec((QBLK, 3), lambda i: (i, 0)),
            pl.BlockSpec((C_IN, C1), lambda i: (0, 0)),
        ],
        out_specs=[
            pl.BlockSpec((1, C1), lambda i: (0, 0)),
            pl.BlockSpec((1, C1), lambda i: (0, 0)),
        ],
        out_shape=[
            jax.ShapeDtypeStruct((1, C1), jnp.float32),
            jax.ShapeDtypeStruct((1, C1), jnp.float32),
        ],
    )(gflat, nxflat, w0t)


def _k4b_body(g_ref, nx_ref, w0t_ref, s1_ref, q1_ref, g0_ref, be0_ref,
              w1t_ref, b1_ref, z2_ref, s2_ref, q2_ref):
    m = jnp.float32(B * S * NS)
    m1 = s1_ref[...] / m
    v1 = q1_ref[...] / m - m1 * m1
    scale = lax.rsqrt(v1 + EPS) * g0_ref[...]
    shift = be0_ref[...] - m1 * scale

    q = jnp.dot(nx_ref[...], w0t_ref[0:3, :], precision=HIGH,
                preferred_element_type=jnp.float32)
    z1 = g_ref[...].reshape(QBLK, NS, C1) - q[:, None, :]
    a1 = jnp.maximum(z1 * scale[None] + shift[None], 0.0)
    z2 = (jnp.dot(a1.reshape(RBLK, C1), w1t_ref[...], precision=HIGH,
                  preferred_element_type=jnp.float32) + b1_ref[...])
    z2_ref[...] = z2
    s = jnp.sum(z2, axis=0).reshape(1, C2)
    sq = jnp.sum(z2 * z2, axis=0).reshape(1, C2)

    @pl.when(pl.program_id(0) == 0)
    def _():
        s2_ref[...] = jnp.zeros_like(s2_ref)
        q2_ref[...] = jnp.zeros_like(q2_ref)

    s2_ref[...] += s
    q2_ref[...] += sq


def _run_k4b(gflat, nxflat, w0t, s1, q1, g0, be0, w1t, b1):
    return pl.pallas_call(
        _k4b_body,
        grid=(NBLK,),
        in_specs=[
            pl.BlockSpec((RBLK, C1), lambda i: (i, 0)),
            pl.BlockSpec((QBLK, 3), lambda i: (i, 0)),
            pl.BlockSpec((C_IN, C1), lambda i: (0, 0)),
            pl.BlockSpec((1, C1), lambda i: (0, 0)),
            pl.BlockSpec((1, C1), lambda i: (0, 0)),
            pl.BlockSpec((1, C1), lambda i: (0, 0)),
            pl.BlockSpec((1, C1), lambda i: (0, 0)),
            pl.BlockSpec((C1, C2), lambda i: (0, 0)),
            pl.BlockSpec((1, C2), lambda i: (0, 0)),
        ],
        out_specs=[
            pl.BlockSpec((RBLK, C2), lambda i: (i, 0)),
            pl.BlockSpec((1, C2), lambda i: (0, 0)),
            pl.BlockSpec((1, C2), lambda i: (0, 0)),
        ],
        out_shape=[
            jax.ShapeDtypeStruct((B * S * NS, C2), jnp.float32),
            jax.ShapeDtypeStruct((1, C2), jnp.float32),
            jax.ShapeDtypeStruct((1, C2), jnp.float32),
        ],
    )(gflat, nxflat, w0t, s1, q1, g0, be0, w1t, b1)


def _k4c_body(z2_ref, s2_ref, q2_ref, g1_ref, be1_ref, out_ref):
    m = jnp.float32(B * S * NS)
    m2 = s2_ref[...] / m
    v2 = q2_ref[...] / m - m2 * m2
    scale = lax.rsqrt(v2 + EPS) * g1_ref[...]
    shift = be1_ref[...] - m2 * scale
    a2 = jnp.maximum(z2_ref[...] * scale + shift, 0.0)
    out_ref[...] = jnp.max(a2.reshape(QBLK, NS, C2), axis=1)


def _run_k4c(z2, s2, q2, g1, be1):
    return pl.pallas_call(
        _k4c_body,
        grid=(NBLK,),
        in_specs=[
            pl.BlockSpec((RBLK, C2), lambda i: (i, 0)),
            pl.BlockSpec((1, C2), lambda i: (0, 0)),
            pl.BlockSpec((1, C2), lambda i: (0, 0)),
            pl.BlockSpec((1, C2), lambda i: (0, 0)),
            pl.BlockSpec((1, C2), lambda i: (0, 0)),
        ],
        out_specs=pl.BlockSpec((QBLK, C2), lambda i: (i, 0)),
        out_shape=jax.ShapeDtypeStruct((B * S, C2), jnp.float32),
    )(z2, s2, q2, g1, be1)


def kernel(xyz, points, idx, conv_w0, conv_b0, bn_g0, bn_b0,
           conv_w1, conv_b1, bn_g1, bn_b1):
    w0t = conv_w0.T
    w1t = conv_w1.T
    b0 = conv_b0.reshape(1, C1)
    b1 = conv_b1.reshape(1, C2)
    g0 = bn_g0.reshape(1, C1)
    be0 = bn_b0.reshape(1, C1)
    g1 = bn_g1.reshape(1, C2)
    be1 = bn_b1.reshape(1, C2)

    f1, gidx = _run_k1(xyz, points, w0t, b0, idx.reshape(B, 1, S))
    xyzs = jnp.transpose(xyz, (0, 2, 1)).reshape(B, 3, NSUB, NLANE)

    xyzp = jnp.pad(xyz.reshape(B * N, 3), ((0, 0), (0, 13)))
    q16 = _make_sc_gather(B * S, 16)(xyzp, gidx.reshape(B * S))
    new_xyz = q16[:, :3].reshape(B, S, 3)

    knn = _run_k2(xyzs, new_xyz)

    gflat = _make_sc_gather(_IDX_TOTAL, C1)(
        f1.reshape(B * N, C1), knn.reshape(_IDX_TOTAL))

    nxflat = new_xyz.reshape(B * S, 3)
    s1, q1 = _run_k4a(gflat, nxflat, w0t)
    z2, s2, q2 = _run_k4b(gflat, nxflat, w0t, s1, q1, g0, be0, w1t, b1)
    out = _run_k4c(z2, s2, q2, g1, be1)
    return (new_xyz, out.reshape(B, S, C2))


# speedup vs baseline: 2.6006x; 2.6006x over previous
import functools

import jax
import jax.numpy as jnp
from jax import lax
from jax.experimental import pallas as pl
from jax.experimental.pallas import tpu as pltpu
from jax.experimental.pallas import tpu_sc as plsc

B, N, S, NS, D = 2, 8192, 2048, 32, 16
C_IN = 3 + D
C1, C2 = 32, 64
EPS = 1e-5
BLK = 128
RBLK = 4096
QBLK = RBLK // NS
NBLK = (B * S * NS) // RBLK
BIG = 3.0e38
HIGH = lax.Precision.HIGHEST


def _k1_body(xyz_ref, pts_ref, w0t_ref, b0_ref, idx_ref, f1_ref, gidx_ref):
    b = pl.program_id(0)
    x = xyz_ref[0]
    p = pts_ref[0]
    w = w0t_ref[...]
    f = (jnp.dot(x, w[0:3, :], precision=HIGH, preferred_element_type=jnp.float32)
         + jnp.dot(p, w[3:, :], precision=HIGH, preferred_element_type=jnp.float32)
         + b0_ref[...])
    f1_ref[0] = f
    gidx_ref[0] = idx_ref[0] + b * N


def _run_k1(xyz, points, w0t, b0, idxr):
    return pl.pallas_call(
        _k1_body,
        grid=(B,),
        in_specs=[
            pl.BlockSpec((1, N, 3), lambda b: (b, 0, 0)),
            pl.BlockSpec((1, N, D), lambda b: (b, 0, 0)),
            pl.BlockSpec((C_IN, C1), lambda b: (0, 0)),
            pl.BlockSpec((1, C1), lambda b: (0, 0)),
            pl.BlockSpec((1, 1, S), lambda b: (b, 0, 0)),
        ],
        out_specs=[
            pl.BlockSpec((1, N, C1), lambda b: (b, 0, 0)),
            pl.BlockSpec((1, 1, S), lambda b: (b, 0, 0)),
        ],
        out_shape=[
            jax.ShapeDtypeStruct((B, N, C1), jnp.float32),
            jax.ShapeDtypeStruct((B, 1, S), jnp.int32),
        ],
    )(xyz, points, w0t, b0, idxr)


FW = 128
RK = 5
EXH = 1.0e37


def _k2_dist(xyzt_ref, q):
    d = None
    for k in range(3):
        pk = xyzt_ref[0, k:k + 1, :]
        diff = q[:, k:k + 1] - pk
        d = diff * diff if d is None else d + diff * diff
    return d


def _fold_min(v, nn):
    w = v.shape[1]
    while w > FW:
        w //= 2
        va, vb = v[:, :w], v[:, w:]
        na, nb = nn[:, :w], nn[:, w:]
        take = (vb < va) | ((vb == va) & (nb < na))
        v = jnp.where(take, vb, va)
        nn = jnp.where(take, nb, na)
    return v, nn


def _k2_body(xyzt_ref, q_ref, knn_ref, hs_ref, js_ref):
    b = pl.program_id(0)
    q = q_ref[0]
    n_iota = lax.broadcasted_iota(jnp.int32, (BLK, N), 1)

    d = _k2_dist(xyzt_ref, q)
    for r in range(RK):
        v, nn = _fold_min(d, n_iota)
        hs_ref[r] = v
        js_ref[r] = nn
        if r < RK - 1:
            parts = []
            for kk in range(N // FW):
                sl = slice(kk * FW, (kk + 1) * FW)
                parts.append(jnp.where(n_iota[:, sl] == nn, BIG, d[:, sl]))
            d = jnp.concatenate(parts, axis=1)

    sus = jnp.zeros((BLK, 1), jnp.int32)
    cols = []
    for _ in range(NS):
        h0 = hs_ref[0]
        j0 = js_ref[0]
        exh = jnp.max((h0 > EXH).astype(jnp.int32), axis=1, keepdims=True)
        sus = jnp.maximum(sus, exh)
        gmin = jnp.min(h0, axis=1, keepdims=True)
        nstar = jnp.min(jnp.where(h0 == gmin, j0, N), axis=1, keepdims=True)
        cols.append(nstar)
        sel = j0 == nstar
        for r in range(RK - 1):
            hs_ref[r] = jnp.where(sel, hs_ref[r + 1], hs_ref[r])
            js_ref[r] = jnp.where(sel, js_ref[r + 1], js_ref[r])
        hs_ref[RK - 1] = jnp.where(sel, BIG, hs_ref[RK - 1])
        js_ref[RK - 1] = jnp.where(sel, N, js_ref[RK - 1])
    knn_ref[0] = jnp.concatenate(cols, axis=1) + b * N

    @pl.when(jnp.max(sus) > 0)
    def _():
        dw = _k2_dist(xyzt_ref, q)
        cols2 = []
        for _ in range(NS):
            g0 = jnp.min(dw, axis=1, keepdims=True)
            n0 = jnp.min(jnp.where(dw == g0, n_iota, N), axis=1, keepdims=True)
            cols2.append(n0)
            dw = jnp.where(n_iota == n0, BIG, dw)
        knn_ref[0] = jnp.concatenate(cols2, axis=1) + b * N


def _run_k2(xyzt, nxyz):
    return pl.pallas_call(
        _k2_body,
        grid=(B, S // BLK),
        in_specs=[
            pl.BlockSpec((1, 3, N), lambda b, j: (b, 0, 0)),
            pl.BlockSpec((1, BLK, 3), lambda b, j: (b, j, 0)),
        ],
        out_specs=pl.BlockSpec((1, BLK, NS), lambda b, j: (b, j, 0)),
        out_shape=jax.ShapeDtypeStruct((B, S, NS), jnp.int32),
        scratch_shapes=[
            pltpu.VMEM((RK, BLK, FW), jnp.float32),
            pltpu.VMEM((RK, BLK, FW), jnp.int32),
        ],
    )(xyzt, nxyz)


_IDX_TOTAL = B * S * NS
_GCH = 128


def _make_sc_gather(n_idx, width):
    info = plsc.get_sparse_core_info()
    nw = info.num_cores * info.num_subcores
    per_w = n_idx // nw
    nch = per_w // _GCH
    mesh = plsc.VectorSubcoreMesh(core_axis_name="c", subcore_axis_name="s")

    @functools.partial(
        pl.kernel,
        mesh=mesh,
        compiler_params=pltpu.CompilerParams(use_tc_tiling_on_sc=False),
        out_type=jax.ShapeDtypeStruct((n_idx, width), jnp.float32),
        scratch_types=[
            pltpu.VMEM((per_w,), jnp.int32),
            pltpu.VMEM((_GCH, width), jnp.float32),
            pltpu.SemaphoreType.DMA,
        ],
    )
    def gather_k(table_hbm, idx_hbm, out_hbm, idx_v, buf0, sem0):
        wid = lax.axis_index("s") * info.num_cores + lax.axis_index("c")
        base = pl.multiple_of(wid * per_w, _GCH)
        pltpu.sync_copy(idx_hbm.at[pl.ds(base, per_w)], idx_v)

        def body(j, _):
            off = pl.multiple_of(j * _GCH, _GCH)
            pltpu.async_copy(
                table_hbm.at[idx_v.at[pl.ds(off, _GCH)]], buf0, sem0).wait()
            dst = pl.multiple_of(base + j * _GCH, _GCH)
            pltpu.sync_copy(buf0, out_hbm.at[pl.ds(dst, _GCH)])
            return 0

        lax.fori_loop(0, nch, body, 0)

    return gather_k


def _k4a_body(g_ref, nx_ref, w0t_ref, s1_ref, q1_ref):
    q = jnp.dot(nx_ref[...], w0t_ref[0:3, :], precision=HIGH,
                preferred_element_type=jnp.float32)
    z1 = g_ref[...].reshape(QBLK, NS, C1) - q[:, None, :]
    s = jnp.sum(z1, axis=(0, 1)).reshape(1, C1)
    sq = jnp.sum(z1 * z1, axis=(0, 1)).reshape(1, C1)

    @pl.when(pl.program_id(0) == 0)
    def _():
        s1_ref[...] = jnp.zeros_like(s1_ref)
        q1_ref[...] = jnp.zeros_like(q1_ref)

    s1_ref[...] += s
    q1_ref[...] += sq


def _run_k4a(gflat, nxflat, w0t):
    return pl.pallas_call(
        _k4a_body,
        grid=(NBLK,),
        in_specs=[
            pl.BlockSpec((RBLK, C1), lambda i: (i, 0)),
            pl.BlockSpec((QBLK, 3), lambda i: (i, 0)),
            pl.BlockSpec((C_IN, C1), lambda i: (0, 0)),
        ],
        out_specs=[
            pl.BlockSpec((1, C1), lambda i: (0, 0)),
            pl.BlockSpec((1, C1), lambda i: (0, 0)),
        ],
        out_shape=[
            jax.ShapeDtypeStruct((1, C1), jnp.float32),
            jax.ShapeDtypeStruct((1, C1), jnp.float32),
        ],
    )(gflat, nxflat, w0t)


def _k4b_body(g_ref, nx_ref, w0t_ref, s1_ref, q1_ref, g0_ref, be0_ref,
              w1t_ref, b1_ref, z2_ref, s2_ref, q2_ref):
    m = jnp.float32(B * S * NS)
    m1 = s1_ref[...] / m
    v1 = q1_ref[...] / m - m1 * m1
    scale = lax.rsqrt(v1 + EPS) * g0_ref[...]
    shift = be0_ref[...] - m1 * scale

    q = jnp.dot(nx_ref[...], w0t_ref[0:3, :], precision=HIGH,
                preferred_element_type=jnp.float32)
    z1 = g_ref[...].reshape(QBLK, NS, C1) - q[:, None, :]
    a1 = jnp.maximum(z1 * scale[None] + shift[None], 0.0)
    z2 = (jnp.dot(a1.reshape(RBLK, C1), w1t_ref[...], precision=HIGH,
                  preferred_element_type=jnp.float32) + b1_ref[...])
    z2_ref[...] = z2
    s = jnp.sum(z2, axis=0).reshape(1, C2)
    sq = jnp.sum(z2 * z2, axis=0).reshape(1, C2)

    @pl.when(pl.program_id(0) == 0)
    def _():
        s2_ref[...] = jnp.zeros_like(s2_ref)
        q2_ref[...] = jnp.zeros_like(q2_ref)

    s2_ref[...] += s
    q2_ref[...] += sq


def _run_k4b(gflat, nxflat, w0t, s1, q1, g0, be0, w1t, b1):
    return pl.pallas_call(
        _k4b_body,
        grid=(NBLK,),
        in_specs=[
            pl.BlockSpec((RBLK, C1), lambda i: (i, 0)),
            pl.BlockSpec((QBLK, 3), lambda i: (i, 0)),
            pl.BlockSpec((C_IN, C1), lambda i: (0, 0)),
            pl.BlockSpec((1, C1), lambda i: (0, 0)),
            pl.BlockSpec((1, C1), lambda i: (0, 0)),
            pl.BlockSpec((1, C1), lambda i: (0, 0)),
            pl.BlockSpec((1, C1), lambda i: (0, 0)),
            pl.BlockSpec((C1, C2), lambda i: (0, 0)),
            pl.BlockSpec((1, C2), lambda i: (0, 0)),
        ],
        out_specs=[
            pl.BlockSpec((RBLK, C2), lambda i: (i, 0)),
            pl.BlockSpec((1, C2), lambda i: (0, 0)),
            pl.BlockSpec((1, C2), lambda i: (0, 0)),
        ],
        out_shape=[
            jax.ShapeDtypeStruct((B * S * NS, C2), jnp.float32),
            jax.ShapeDtypeStruct((1, C2), jnp.float32),
            jax.ShapeDtypeStruct((1, C2), jnp.float32),
        ],
    )(gflat, nxflat, w0t, s1, q1, g0, be0, w1t, b1)


def _k4c_body(z2_ref, s2_ref, q2_ref, g1_ref, be1_ref, out_ref):
    m = jnp.float32(B * S * NS)
    m2 = s2_ref[...] / m
    v2 = q2_ref[...] / m - m2 * m2
    scale = lax.rsqrt(v2 + EPS) * g1_ref[...]
    shift = be1_ref[...] - m2 * scale
    a2 = jnp.maximum(z2_ref[...] * scale + shift, 0.0)
    out_ref[...] = jnp.max(a2.reshape(QBLK, NS, C2), axis=1)


def _run_k4c(z2, s2, q2, g1, be1):
    return pl.pallas_call(
        _k4c_body,
        grid=(NBLK,),
        in_specs=[
            pl.BlockSpec((RBLK, C2), lambda i: (i, 0)),
            pl.BlockSpec((1, C2), lambda i: (0, 0)),
            pl.BlockSpec((1, C2), lambda i: (0, 0)),
            pl.BlockSpec((1, C2), lambda i: (0, 0)),
            pl.BlockSpec((1, C2), lambda i: (0, 0)),
        ],
        out_specs=pl.BlockSpec((QBLK, C2), lambda i: (i, 0)),
        out_shape=jax.ShapeDtypeStruct((B * S, C2), jnp.float32),
    )(z2, s2, q2, g1, be1)


def kernel(xyz, points, idx, conv_w0, conv_b0, bn_g0, bn_b0,
           conv_w1, conv_b1, bn_g1, bn_b1):
    w0t = conv_w0.T
    w1t = conv_w1.T
    b0 = conv_b0.reshape(1, C1)
    b1 = conv_b1.reshape(1, C2)
    g0 = bn_g0.reshape(1, C1)
    be0 = bn_b0.reshape(1, C1)
    g1 = bn_g1.reshape(1, C2)
    be1 = bn_b1.reshape(1, C2)

    f1, gidx = _run_k1(xyz, points, w0t, b0, idx.reshape(B, 1, S))
    xyzt = jnp.transpose(xyz, (0, 2, 1))

    xyzp = jnp.pad(xyz.reshape(B * N, 3), ((0, 0), (0, 13)))
    q16 = _make_sc_gather(B * S, 16)(xyzp, gidx.reshape(B * S))
    new_xyz = q16[:, :3].reshape(B, S, 3)

    knn = _run_k2(xyzt, new_xyz)

    gflat = _make_sc_gather(_IDX_TOTAL, C1)(
        f1.reshape(B * N, C1), knn.reshape(_IDX_TOTAL))

    nxflat = new_xyz.reshape(B * S, 3)
    s1, q1 = _run_k4a(gflat, nxflat, w0t)
    z2, s2, q2 = _run_k4b(gflat, nxflat, w0t, s1, q1, g0, be0, w1t, b1)
    out = _run_k4c(z2, s2, q2, g1, be1)
    return (new_xyz, out.reshape(B, S, C2))


# speedup vs baseline: 2.8032x; 1.0779x over previous
import functools

import jax
import jax.numpy as jnp
from jax import lax
from jax.experimental import pallas as pl
from jax.experimental.pallas import tpu as pltpu
from jax.experimental.pallas import tpu_sc as plsc

B, N, S, NS, D = 2, 8192, 2048, 32, 16
C_IN = 3 + D
C1, C2 = 32, 64
EPS = 1e-5
BLK = 128
RBLK = 4096
QBLK = RBLK // NS
NBLK = (B * S * NS) // RBLK
BIG = 3.0e38
HIGH = lax.Precision.HIGHEST


def _k1_body(xyz_ref, pts_ref, w0t_ref, b0_ref, idx_ref, f1_ref, gidx_ref):
    b = pl.program_id(0)
    x = xyz_ref[0]
    p = pts_ref[0]
    w = w0t_ref[...]
    f = (jnp.dot(x, w[0:3, :], precision=HIGH, preferred_element_type=jnp.float32)
         + jnp.dot(p, w[3:, :], precision=HIGH, preferred_element_type=jnp.float32)
         + b0_ref[...])
    f1_ref[0] = f
    gidx_ref[0] = idx_ref[0] + b * N


def _run_k1(xyz, points, w0t, b0, idxr):
    return pl.pallas_call(
        _k1_body,
        grid=(B,),
        in_specs=[
            pl.BlockSpec((1, N, 3), lambda b: (b, 0, 0)),
            pl.BlockSpec((1, N, D), lambda b: (b, 0, 0)),
            pl.BlockSpec((C_IN, C1), lambda b: (0, 0)),
            pl.BlockSpec((1, C1), lambda b: (0, 0)),
            pl.BlockSpec((1, 1, S), lambda b: (b, 0, 0)),
        ],
        out_specs=[
            pl.BlockSpec((1, N, C1), lambda b: (b, 0, 0)),
            pl.BlockSpec((1, 1, S), lambda b: (b, 0, 0)),
        ],
        out_shape=[
            jax.ShapeDtypeStruct((B, N, C1), jnp.float32),
            jax.ShapeDtypeStruct((B, 1, S), jnp.int32),
        ],
    )(xyz, points, w0t, b0, idxr)


FW = 128
RK = 5
EXH = 1.0e37


def _k2_dist(xyzt_ref, q):
    d = None
    for k in range(3):
        pk = xyzt_ref[0, k:k + 1, :]
        diff = q[:, k:k + 1] - pk
        d = diff * diff if d is None else d + diff * diff
    return d


def _fold_min(v, nn):
    w = v.shape[1]
    while w > FW:
        w //= 2
        take = v[:, w:] < v[:, :w]
        nn = jnp.where(take, nn[:, w:], nn[:, :w])
        v = jnp.where(take, v[:, w:], v[:, :w])
    return v, nn


def _k2_body(xyzt_ref, q_ref, knn_ref, hs_ref, js_ref):
    b = pl.program_id(0)
    q = q_ref[0]
    p_iota = lax.broadcasted_iota(jnp.int32, (BLK, N), 1)
    n_iota = (p_iota % FW) * (N // FW) + p_iota // FW

    d = _k2_dist(xyzt_ref, q)
    for r in range(RK):
        v, nn = _fold_min(d, n_iota)
        hs_ref[r] = v
        js_ref[r] = nn
        if r < RK - 1:
            parts = []
            for kk in range(N // FW):
                sl = slice(kk * FW, (kk + 1) * FW)
                parts.append(jnp.where(n_iota[:, sl] == nn, BIG, d[:, sl]))
            d = jnp.concatenate(parts, axis=1)

    sus = jnp.zeros((BLK, 1), jnp.int32)
    cols = []
    for _ in range(NS):
        h0 = hs_ref[0]
        j0 = js_ref[0]
        exh = jnp.max((h0 > EXH).astype(jnp.int32), axis=1, keepdims=True)
        sus = jnp.maximum(sus, exh)
        gmin = jnp.min(h0, axis=1, keepdims=True)
        nstar = jnp.min(jnp.where(h0 == gmin, j0, N), axis=1, keepdims=True)
        cols.append(nstar)
        sel = j0 == nstar
        for r in range(RK - 1):
            hs_ref[r] = jnp.where(sel, hs_ref[r + 1], hs_ref[r])
            js_ref[r] = jnp.where(sel, js_ref[r + 1], js_ref[r])
        hs_ref[RK - 1] = jnp.where(sel, BIG, hs_ref[RK - 1])
        js_ref[RK - 1] = jnp.where(sel, N, js_ref[RK - 1])
    knn_ref[0] = jnp.concatenate(cols, axis=1) + b * N

    @pl.when(jnp.max(sus) > 0)
    def _():
        dw = _k2_dist(xyzt_ref, q)
        cols2 = []
        for _ in range(NS):
            g0 = jnp.min(dw, axis=1, keepdims=True)
            n0 = jnp.min(jnp.where(dw == g0, n_iota, N), axis=1, keepdims=True)
            cols2.append(n0)
            dw = jnp.where(n_iota == n0, BIG, dw)
        knn_ref[0] = jnp.concatenate(cols2, axis=1) + b * N


def _run_k2(xyzt, nxyz):
    return pl.pallas_call(
        _k2_body,
        grid=(B, S // BLK),
        in_specs=[
            pl.BlockSpec((1, 3, N), lambda b, j: (b, 0, 0)),
            pl.BlockSpec((1, BLK, 3), lambda b, j: (b, j, 0)),
        ],
        out_specs=pl.BlockSpec((1, BLK, NS), lambda b, j: (b, j, 0)),
        out_shape=jax.ShapeDtypeStruct((B, S, NS), jnp.int32),
        scratch_shapes=[
            pltpu.VMEM((RK, BLK, FW), jnp.float32),
            pltpu.VMEM((RK, BLK, FW), jnp.int32),
        ],
    )(xyzt, nxyz)


_IDX_TOTAL = B * S * NS
_GCH = 128


def _make_sc_gather(n_idx, width):
    info = plsc.get_sparse_core_info()
    nw = info.num_cores * info.num_subcores
    per_w = n_idx // nw
    nch = per_w // _GCH
    mesh = plsc.VectorSubcoreMesh(core_axis_name="c", subcore_axis_name="s")

    @functools.partial(
        pl.kernel,
        mesh=mesh,
        compiler_params=pltpu.CompilerParams(use_tc_tiling_on_sc=False),
        out_type=jax.ShapeDtypeStruct((n_idx, width), jnp.float32),
        scratch_types=[
            pltpu.VMEM((per_w,), jnp.int32),
            pltpu.VMEM((_GCH, width), jnp.float32),
            pltpu.SemaphoreType.DMA,
        ],
    )
    def gather_k(table_hbm, idx_hbm, out_hbm, idx_v, buf0, sem0):
        wid = lax.axis_index("s") * info.num_cores + lax.axis_index("c")
        base = pl.multiple_of(wid * per_w, _GCH)
        pltpu.sync_copy(idx_hbm.at[pl.ds(base, per_w)], idx_v)

        def body(j, _):
            off = pl.multiple_of(j * _GCH, _GCH)
            pltpu.async_copy(
                table_hbm.at[idx_v.at[pl.ds(off, _GCH)]], buf0, sem0).wait()
            dst = pl.multiple_of(base + j * _GCH, _GCH)
            pltpu.sync_copy(buf0, out_hbm.at[pl.ds(dst, _GCH)])
            return 0

        lax.fori_loop(0, nch, body, 0)

    return gather_k


def _k4a_body(g_ref, nx_ref, w0t_ref, s1_ref, q1_ref):
    q = jnp.dot(nx_ref[...], w0t_ref[0:3, :], precision=HIGH,
                preferred_element_type=jnp.float32)
    z1 = g_ref[...].reshape(QBLK, NS, C1) - q[:, None, :]
    s = jnp.sum(z1, axis=(0, 1)).reshape(1, C1)
    sq = jnp.sum(z1 * z1, axis=(0, 1)).reshape(1, C1)

    @pl.when(pl.program_id(0) == 0)
    def _():
        s1_ref[...] = jnp.zeros_like(s1_ref)
        q1_ref[...] = jnp.zeros_like(q1_ref)

    s1_ref[...] += s
    q1_ref[...] += sq


def _run_k4a(gflat, nxflat, w0t):
    return pl.pallas_call(
        _k4a_body,
        grid=(NBLK,),
        in_specs=[
            pl.BlockSpec((RBLK, C1), lambda i: (i, 0)),
            pl.BlockSpec((QBLK, 3), lambda i: (i, 0)),
            pl.BlockSpec((C_IN, C1), lambda i: (0, 0)),
        ],
        out_specs=[
            pl.BlockSpec((1, C1), lambda i: (0, 0)),
            pl.BlockSpec((1, C1), lambda i: (0, 0)),
        ],
        out_shape=[
            jax.ShapeDtypeStruct((1, C1), jnp.float32),
            jax.ShapeDtypeStruct((1, C1), jnp.float32),
        ],
    )(gflat, nxflat, w0t)


def _k4b_body(g_ref, nx_ref, w0t_ref, s1_ref, q1_ref, g0_ref, be0_ref,
              w1t_ref, b1_ref, z2_ref, s2_ref, q2_ref):
    m = jnp.float32(B * S * NS)
    m1 = s1_ref[...] / m
    v1 = q1_ref[...] / m - m1 * m1
    scale = lax.rsqrt(v1 + EPS) * g0_ref[...]
    shift = be0_ref[...] - m1 * scale

    q = jnp.dot(nx_ref[...], w0t_ref[0:3, :], precision=HIGH,
                preferred_element_type=jnp.float32)
    z1 = g_ref[...].reshape(QBLK, NS, C1) - q[:, None, :]
    a1 = jnp.maximum(z1 * scale[None] + shift[None], 0.0)
    z2 = (jnp.dot(a1.reshape(RBLK, C1), w1t_ref[...], precision=HIGH,
                  preferred_element_type=jnp.float32) + b1_ref[...])
    z2_ref[...] = z2
    s = jnp.sum(z2, axis=0).reshape(1, C2)
    sq = jnp.sum(z2 * z2, axis=0).reshape(1, C2)

    @pl.when(pl.program_id(0) == 0)
    def _():
        s2_ref[...] = jnp.zeros_like(s2_ref)
        q2_ref[...] = jnp.zeros_like(q2_ref)

    s2_ref[...] += s
    q2_ref[...] += sq


def _run_k4b(gflat, nxflat, w0t, s1, q1, g0, be0, w1t, b1):
    return pl.pallas_call(
        _k4b_body,
        grid=(NBLK,),
        in_specs=[
            pl.BlockSpec((RBLK, C1), lambda i: (i, 0)),
            pl.BlockSpec((QBLK, 3), lambda i: (i, 0)),
            pl.BlockSpec((C_IN, C1), lambda i: (0, 0)),
            pl.BlockSpec((1, C1), lambda i: (0, 0)),
            pl.BlockSpec((1, C1), lambda i: (0, 0)),
            pl.BlockSpec((1, C1), lambda i: (0, 0)),
            pl.BlockSpec((1, C1), lambda i: (0, 0)),
            pl.BlockSpec((C1, C2), lambda i: (0, 0)),
            pl.BlockSpec((1, C2), lambda i: (0, 0)),
        ],
        out_specs=[
            pl.BlockSpec((RBLK, C2), lambda i: (i, 0)),
            pl.BlockSpec((1, C2), lambda i: (0, 0)),
            pl.BlockSpec((1, C2), lambda i: (0, 0)),
        ],
        out_shape=[
            jax.ShapeDtypeStruct((B * S * NS, C2), jnp.float32),
            jax.ShapeDtypeStruct((1, C2), jnp.float32),
            jax.ShapeDtypeStruct((1, C2), jnp.float32),
        ],
    )(gflat, nxflat, w0t, s1, q1, g0, be0, w1t, b1)


def _k4c_body(z2_ref, s2_ref, q2_ref, g1_ref, be1_ref, out_ref):
    m = jnp.float32(B * S * NS)
    m2 = s2_ref[...] / m
    v2 = q2_ref[...] / m - m2 * m2
    scale = lax.rsqrt(v2 + EPS) * g1_ref[...]
    shift = be1_ref[...] - m2 * scale
    a2 = jnp.maximum(z2_ref[...] * scale + shift, 0.0)
    out_ref[...] = jnp.max(a2.reshape(QBLK, NS, C2), axis=1)


def _run_k4c(z2, s2, q2, g1, be1):
    return pl.pallas_call(
        _k4c_body,
        grid=(NBLK,),
        in_specs=[
            pl.BlockSpec((RBLK, C2), lambda i: (i, 0)),
            pl.BlockSpec((1, C2), lambda i: (0, 0)),
            pl.BlockSpec((1, C2), lambda i: (0, 0)),
            pl.BlockSpec((1, C2), lambda i: (0, 0)),
            pl.BlockSpec((1, C2), lambda i: (0, 0)),
        ],
        out_specs=pl.BlockSpec((QBLK, C2), lambda i: (i, 0)),
        out_shape=jax.ShapeDtypeStruct((B * S, C2), jnp.float32),
    )(z2, s2, q2, g1, be1)


def kernel(xyz, points, idx, conv_w0, conv_b0, bn_g0, bn_b0,
           conv_w1, conv_b1, bn_g1, bn_b1):
    w0t = conv_w0.T
    w1t = conv_w1.T
    b0 = conv_b0.reshape(1, C1)
    b1 = conv_b1.reshape(1, C2)
    g0 = bn_g0.reshape(1, C1)
    be0 = bn_b0.reshape(1, C1)
    g1 = bn_g1.reshape(1, C2)
    be1 = bn_b1.reshape(1, C2)

    f1, gidx = _run_k1(xyz, points, w0t, b0, idx.reshape(B, 1, S))
    xyzt = (jnp.transpose(xyz, (0, 2, 1))
            .reshape(B, 3, FW, N // FW).swapaxes(2, 3).reshape(B, 3, N))

    xyzp = jnp.pad(xyz.reshape(B * N, 3), ((0, 0), (0, 13)))
    q16 = _make_sc_gather(B * S, 16)(xyzp, gidx.reshape(B * S))
    new_xyz = q16[:, :3].reshape(B, S, 3)

    knn = _run_k2(xyzt, new_xyz)

    gflat = _make_sc_gather(_IDX_TOTAL, C1)(
        f1.reshape(B * N, C1), knn.reshape(_IDX_TOTAL))

    nxflat = new_xyz.reshape(B * S, 3)
    s1, q1 = _run_k4a(gflat, nxflat, w0t)
    z2, s2, q2 = _run_k4b(gflat, nxflat, w0t, s1, q1, g0, be0, w1t, b1)
    out = _run_k4c(z2, s2, q2, g1, be1)
    return (new_xyz, out.reshape(B, S, C2))
